# baseline (speedup 1.0000x reference)
import jax
import jax.numpy as jnp
from jax.experimental import pallas as pl

_EMB = 64
_POS = 588
_DIN = 16
_B = 1024
_NCH = 8

_G = 4
_PG = _POS // _G
_KP = _G * _DIN
_NP = _G * _EMB
_KA = _KP + _G * _NCH

_BB = 16
_RB = _BB * _PG
_ROWS = _B * _PG


def _kernel_body(cb_ref, w_ref, b_ref, x_ref, o_ref):
    iota = jax.lax.broadcasted_iota(jnp.int32, (_RB, _NCH), 1)
    oh = jnp.concatenate(
        [(cb_ref[:, g][:, None] == iota).astype(jnp.float32) for g in range(_G)],
        axis=1)
    xa = jnp.concatenate([x_ref[...], oh], axis=1)
    d = jnp.dot(xa, w_ref[...], preferred_element_type=jnp.float32)
    o_ref[...] = d + b_ref[0, :]


def kernel(x, emb_table, W, b, channel_base):
    xg = x.reshape(_ROWS, _KP)
    eye = jnp.eye(_G, dtype=jnp.float32)
    Wg = jnp.kron(eye, W)
    embg = jnp.kron(eye, emb_table)
    Wfull = jnp.concatenate([Wg, embg], axis=0)
    bg = jnp.tile(b, _G).reshape(1, _NP)
    cbT = jnp.tile(channel_base.astype(jnp.int32).reshape(_PG, _G), (_BB, 1))
    grid = (_ROWS // _RB,)
    out = pl.pallas_call(
        _kernel_body,
        grid=grid,
        in_specs=[
            pl.BlockSpec((_RB, _G), lambda i: (0, 0)),
            pl.BlockSpec((_KA, _NP), lambda i: (0, 0)),
            pl.BlockSpec((1, _NP), lambda i: (0, 0)),
            pl.BlockSpec((_RB, _KP), lambda i: (i, 0)),
        ],
        out_specs=pl.BlockSpec((_RB, _NP), lambda i: (i, 0)),
        out_shape=jax.ShapeDtypeStruct((_ROWS, _NP), jnp.float32),
    )(cbT, Wfull, bg, xg)
    return out.reshape(_B, _POS, _EMB)


# speedup vs baseline: 1.1347x; 1.1347x over previous
import jax
import jax.numpy as jnp
from jax.experimental import pallas as pl
from jax.experimental.pallas import tpu as pltpu

_EMB = 64
_POS = 588
_DIN = 16
_B = 1024
_NCH = 8

_G = 4
_PG = _POS // _G
_KP = _G * _DIN
_NP = _G * _EMB

_BB = 32
_RB = _BB * _PG
_ROWS = _B * _PG


def _kernel_body(cb_ref, emb_ref, w_ref, b_ref, x_ref, o_ref, y_scr):
    @pl.when(pl.program_id(0) == 0)
    def _init():
        iota = jax.lax.broadcasted_iota(jnp.int32, (_RB, _NCH), 1)
        oh = jnp.concatenate(
            [(cb_ref[:, g][:, None] == iota).astype(jnp.float32)
             for g in range(_G)], axis=1)
        y = jnp.dot(oh, emb_ref[...], preferred_element_type=jnp.float32)
        y_scr[...] = y + b_ref[0, :]

    d = jnp.dot(x_ref[...], w_ref[...], preferred_element_type=jnp.float32)
    o_ref[...] = d + y_scr[...]


def kernel(x, emb_table, W, b, channel_base):
    xg = x.reshape(_ROWS, _KP)
    eye = jnp.eye(_G, dtype=jnp.float32)
    Wg = jnp.kron(eye, W)
    embg = jnp.kron(eye, emb_table)
    bg = jnp.tile(b, _G).reshape(1, _NP)
    cbT = jnp.tile(channel_base.astype(jnp.int32).reshape(_PG, _G), (_BB, 1))
    grid = (_ROWS // _RB,)
    out = pl.pallas_call(
        _kernel_body,
        grid=grid,
        in_specs=[
            pl.BlockSpec((_RB, _G), lambda i: (0, 0)),
            pl.BlockSpec((_G * _NCH, _NP), lambda i: (0, 0)),
            pl.BlockSpec((_KP, _NP), lambda i: (0, 0)),
            pl.BlockSpec((1, _NP), lambda i: (0, 0)),
            pl.BlockSpec((_RB, _KP), lambda i: (i, 0)),
        ],
        out_specs=pl.BlockSpec((_RB, _NP), lambda i: (i, 0)),
        out_shape=jax.ShapeDtypeStruct((_ROWS, _NP), jnp.float32),
        scratch_shapes=[pltpu.VMEM((_RB, _NP), jnp.float32)],
    )(cbT, embg, Wg, bg, xg)
    return out.reshape(_B, _POS, _EMB)


# speedup vs baseline: 1.2258x; 1.0803x over previous
import jax
import jax.numpy as jnp
from jax.experimental import pallas as pl

_EMB = 64
_POS = 588
_DIN = 16
_B = 1024
_NCH = 8

_BB = 32


def _kernel_body(cb_ref, w_ref, x_ref, o_ref):
    iota = jax.lax.broadcasted_iota(jnp.int32, (_POS, _NCH), 1)
    oh = (cb_ref[...] == iota).astype(jnp.bfloat16)
    wb = w_ref[...].astype(jnp.bfloat16)
    for bb in range(_BB):
        xa = jnp.concatenate([x_ref[bb].astype(jnp.bfloat16), oh], axis=1)
        o_ref[bb] = jnp.dot(xa, wb, preferred_element_type=jnp.float32)


def kernel(x, emb_table, W, b, channel_base):
    cb = channel_base.astype(jnp.int32).reshape(_POS, 1)
    Wfull = jnp.concatenate([W, emb_table + b[None, :]], axis=0)
    grid = (_B // _BB,)
    return pl.pallas_call(
        _kernel_body,
        grid=grid,
        in_specs=[
            pl.BlockSpec((_POS, 1), lambda i: (0, 0)),
            pl.BlockSpec((_DIN + _NCH, _EMB), lambda i: (0, 0)),
            pl.BlockSpec((_BB, _POS, _DIN), lambda i: (i, 0, 0)),
        ],
        out_specs=pl.BlockSpec((_BB, _POS, _EMB), lambda i: (i, 0, 0)),
        out_shape=jax.ShapeDtypeStruct((_B, _POS, _EMB), jnp.float32),
    )(cb, Wfull, x)
